# baseline (speedup 1.0000x reference)
import functools

import jax
import jax.numpy as jnp
from jax import lax
from jax.experimental import pallas as pl
from jax.experimental.pallas import tpu as pltpu
from jax.experimental.pallas import tpu_sc as plsc

D = 32
L = 16
NC, NS = 2, 16
NW = NC * NS
R = 32

K_GENRE, K_DIR, K_ACTOR = 8, 5, 20
SEG_RATE = 0
SEG_GENRE = R
SEG_DIR = SEG_GENRE + R * K_GENRE
SEG_ACTOR = SEG_DIR + R * K_DIR
NPR = SEG_ACTOR + R * K_ACTOR
POOLED = ((0, SEG_GENRE, K_GENRE), (1, SEG_DIR, K_DIR), (2, SEG_ACTOR, K_ACTOR))


@functools.lru_cache(maxsize=None)
def _build(B: int):
  rows_per_w = B // NW
  n_chunks = rows_per_w // R
  assert n_chunks % 2 == 0
  mesh = plsc.VectorSubcoreMesh(core_axis_name="c", subcore_axis_name="s",
                                num_cores=NC, num_subcores=NS)

  @functools.partial(
      pl.kernel,
      out_type=jax.ShapeDtypeStruct((B, 4 * D), jnp.float32),
      mesh=mesh,
      compiler_params=pltpu.CompilerParams(use_tc_tiling_on_sc=False,
                                           needs_layout_passes=False),
      scratch_types=[
          pltpu.VMEM((2, NPR, D), jnp.float32),
          pltpu.VMEM((2, R), jnp.int32),
          pltpu.VMEM((2, R, K_GENRE), jnp.int32),
          pltpu.VMEM((2, R, K_DIR), jnp.int32),
          pltpu.VMEM((2, R, K_ACTOR), jnp.int32),
          pltpu.VMEM((2, NPR), jnp.int32),
          pltpu.VMEM((2, 3, R), jnp.float32),
          pltpu.VMEM((2, 3, R), jnp.float32),
          pltpu.VMEM((3, D), jnp.float32),
          pltpu.VMEM((2, R, 4 * D), jnp.float32),
          pltpu.SemaphoreType.DMA,
          pltpu.SemaphoreType.DMA,
          pltpu.SemaphoreType.DMA,
          pltpu.SemaphoreType.DMA,
          pltpu.SemaphoreType.DMA,
          pltpu.SemaphoreType.DMA,
      ],
  )
  def sc_kernel(ri_hbm, gi_hbm, di_hbm, ai_hbm,
                tr_hbm, tg_hbm, td_hbm, ta_hbm, out_hbm,
                gbuf, idxr, idxg, idxd, idxa, ilist, ab, ab2, t0, outb,
                g0, g1, i0, i1, o0, o1):
    wid = lax.axis_index("c") * NS + lax.axis_index("s")
    gsem = (g0, g1)
    isem = (i0, i1)
    osem = (o0, o1)

    pltpu.sync_copy(tg_hbm.at[pl.ds(0, 1)], t0.at[pl.ds(0, 1)])
    pltpu.sync_copy(td_hbm.at[pl.ds(0, 1)], t0.at[pl.ds(1, 1)])
    pltpu.sync_copy(ta_hbm.at[pl.ds(0, 1)], t0.at[pl.ds(2, 1)])
    t0v = [[t0[fi, pl.ds(h * L, L)] for h in range(2)] for fi in range(3)]

    idx_srcs = ((ri_hbm, idxr), (gi_hbm, idxg), (di_hbm, idxd), (ai_hbm, idxa))

    def stage_idx(c, s, sem):
      base = wid * rows_per_w + c * R
      for src, dst in idx_srcs:
        pltpu.async_copy(src.at[pl.ds(base, R)], dst.at[s], sem)

    def wait_idx(s, sem):
      for src, dst in idx_srcs:
        pltpu.make_async_copy(src.at[pl.ds(0, R)], dst.at[s], sem).wait()

    viota = lax.iota(jnp.int32, L)

    def build_ilist(s):
      for g in range(R // L):
        rows = viota + g * L
        ilist[s, pl.ds(SEG_RATE + g * L, L)] = plsc.load_gather(
            idxr.at[s], [rows])
        for (_, seg, k), src in zip(POOLED, (idxg, idxd, idxa)):
          for j in range(k):
            ilist[s, pl.ds(seg + j * R + g * L, L)] = plsc.load_gather(
                src.at[s], [rows, jnp.full((L,), j, jnp.int32)])

    gplan = (
        (tr_hbm, SEG_RATE, 1),
        (tg_hbm, SEG_GENRE, K_GENRE),
        (td_hbm, SEG_DIR, K_DIR),
        (ta_hbm, SEG_ACTOR, K_ACTOR),
    )
    GMAX = 128

    def fire_gathers(s, sem):
      for tbl, seg, k in gplan:
        off = 0
        while off < k * R:
          n = min(GMAX, k * R - off)
          pltpu.async_copy(tbl.at[ilist.at[s, pl.ds(seg + off, n)]],
                           gbuf.at[s, pl.ds(seg + off, n)], sem)
          off += n

    def wait_gathers(s, sem):
      pltpu.make_async_copy(ta_hbm.at[pl.ds(0, NPR)], gbuf.at[s], sem).wait()

    def phase(c, s):
      o = 1 - s
      base = wid * rows_per_w + c * R

      for fi, seg, k in POOLED:
        for g in range(R // L):
          sl = pl.ds(g * L, L)
          cnt = jnp.zeros((L,), jnp.float32)
          for j in range(k):
            v = ilist[s, pl.ds(seg + j * R + g * L, L)]
            cnt = cnt + jnp.where(v > 0, jnp.float32(1.0), jnp.float32(0.0))
          a = jnp.float32(1.0) / (cnt + jnp.float32(1e-8))
          ab[s, fi, sl] = a
          ab2[s, fi, sl] = (jnp.float32(k) - cnt) * a

      @pl.when(c + 1 < n_chunks)
      def _():
        wait_idx(o, isem[o])
        build_ilist(o)
        fire_gathers(o, gsem[o])

      wait_gathers(s, gsem[s])

      @pl.when(c + 2 < n_chunks)
      def _():
        stage_idx(c + 2, s, isem[s])

      @pl.when(c >= 2)
      def _():
        pltpu.make_async_copy(out_hbm.at[pl.ds(0, R)], outb.at[s],
                              osem[s]).wait()

      def row(r, carry):
        rsp = jnp.full((L,), r, jnp.int32)
        for h in range(2):
          sl = pl.ds(h * L, L)
          outb[s, r, pl.ds(h * L, L)] = gbuf[s, SEG_RATE + r, sl]
        for fi, seg, k in POOLED:
          av = plsc.load_gather(ab.at[s, fi], [rsp])
          bv = plsc.load_gather(ab2.at[s, fi], [rsp])
          for h in range(2):
            sl = pl.ds(h * L, L)
            acc = gbuf[s, seg + r, sl]
            for j in range(1, k):
              acc = acc + gbuf[s, seg + j * R + r, sl]
            outb[s, r, pl.ds((fi + 1) * D + h * L, L)] = (
                acc * av - t0v[fi][h] * bv)
        return carry

      lax.fori_loop(0, R, row, 0)

      pltpu.async_copy(outb.at[s], out_hbm.at[pl.ds(base, R)], osem[s])

    stage_idx(0, 0, isem[0])
    wait_idx(0, isem[0])
    build_ilist(0)
    fire_gathers(0, gsem[0])
    stage_idx(1, 1, isem[1])

    def pair(i, carry):
      phase(2 * i, 0)
      phase(2 * i + 1, 1)
      return carry

    lax.fori_loop(0, n_chunks // 2, pair, 0)

    for s in range(2):
      pltpu.make_async_copy(out_hbm.at[pl.ds(0, R)], outb.at[s],
                            osem[s]).wait()

  return sc_kernel


def kernel(rate_idx, genre_idx, director_idx, actors_idx,
           table_rate, table_genre, table_director, table_actor):
  B = rate_idx.shape[0]
  return _build(B)(rate_idx, genre_idx, director_idx, actors_idx,
                   table_rate, table_genre, table_director, table_actor)


# speedup vs baseline: 1.0107x; 1.0107x over previous
import functools

import jax
import jax.numpy as jnp
from jax import lax
from jax.experimental import pallas as pl
from jax.experimental.pallas import tpu as pltpu
from jax.experimental.pallas import tpu_sc as plsc

D = 32
L = 16
NC, NS = 2, 16
NW = NC * NS
R = 32

K_GENRE, K_DIR, K_ACTOR = 8, 5, 20

CH_ROWS = 10
CH = CH_ROWS * 128
PSEG_RATE = 0
PSEG_GENRE = 128
PSEG_DIR = 384
PSEG_ACTOR = 640

GSEG_RATE = 0
GSEG_GENRE = R
GSEG_DIR = GSEG_GENRE + R * K_GENRE
GSEG_ACTOR = GSEG_DIR + R * K_DIR
NPR = GSEG_ACTOR + R * K_ACTOR

POOLED = ((0, PSEG_GENRE, GSEG_GENRE, K_GENRE),
          (1, PSEG_DIR, GSEG_DIR, K_DIR),
          (2, PSEG_ACTOR, GSEG_ACTOR, K_ACTOR))

GPIECES = ((0, PSEG_RATE, GSEG_RATE, 32),
           (1, PSEG_GENRE, GSEG_GENRE, 128),
           (1, PSEG_GENRE + 128, GSEG_GENRE + 128, 128),
           (2, PSEG_DIR, GSEG_DIR, 128),
           (2, PSEG_DIR + 128, GSEG_DIR + 128, 32),
           *(((3, PSEG_ACTOR + i * 128, GSEG_ACTOR + i * 128, 128)
              for i in range(5))))


def _mesh():
  return plsc.VectorSubcoreMesh(core_axis_name="c", subcore_axis_name="s",
                                num_cores=NC, num_subcores=NS)


@functools.lru_cache(maxsize=None)
def _build_reformat(B: int):
  rows_per_w = B // NW
  n_chunks = rows_per_w // R
  il_rows_w = n_chunks * CH_ROWS

  @functools.partial(
      pl.kernel,
      out_type=jax.ShapeDtypeStruct((NW * il_rows_w, 128), jnp.int32),
      mesh=_mesh(),
      compiler_params=pltpu.CompilerParams(use_tc_tiling_on_sc=True,
                                           needs_layout_passes=False),
      scratch_types=[
          pltpu.VMEM((2, 32), jnp.int32),
          pltpu.VMEM((2, 32, K_GENRE), jnp.int32),
          pltpu.VMEM((2, 32, K_DIR), jnp.int32),
          pltpu.VMEM((2, 32, K_ACTOR), jnp.int32),
          pltpu.VMEM((8 * CH_ROWS, 128), jnp.int32),
          pltpu.SemaphoreType.DMA,
          pltpu.SemaphoreType.DMA,
      ],
  )
  def reformat(ri, gi, di, ai, out_hbm, br, bg, bd, ba, fl, sm0, sm1):
    wid = lax.axis_index("c") * NS + lax.axis_index("s")
    base = wid * rows_per_w
    sems = (sm0, sm1)
    srcs = (ri, gi, di, ai)
    bufs = (br, bg, bd, ba)

    def stage(blk, s):
      for src_, buf in zip(srcs, bufs):
        pltpu.async_copy(src_.at[pl.ds(base + blk * R, R)], buf.at[s],
                         sems[s])

    def wait_stage(s):
      for src_, buf in zip(srcs, bufs):
        pltpu.make_async_copy(src_.at[pl.ds(0, R)], buf.at[s],
                              sems[s]).wait()

    viota = lax.iota(jnp.int32, L)

    def transpose_chunk(blk_in_half, s):
      cbase = blk_in_half * CH
      for g in range(R // L):
        rows = viota + g * L
        roff = g * L

        def put(pos, vec):
          fl[pos // 128, pl.ds(pos % 128, L)] = vec

        put(cbase + PSEG_RATE + roff, br[s, pl.ds(g * L, L)])
        for _, pseg, _, k in POOLED:
          buf = {K_GENRE: bg, K_DIR: bd, K_ACTOR: ba}[k]
          for j in range(k):
            put(cbase + pseg + j * R + roff,
                plsc.load_gather(buf.at[s],
                                 [rows, jnp.full((L,), j, jnp.int32)]))

    stage(0, 0)

    def half(hh, carry0):
      def pairb(i, carry):
        for t in range(2):
          blk_in_half = 2 * i + t
          blk = hh * (n_chunks // 2) + blk_in_half
          s = t
          @pl.when(blk + 1 < n_chunks)
          def _():
            stage(blk + 1, 1 - s)
          wait_stage(s)
          transpose_chunk(blk_in_half, s)
        return carry

      lax.fori_loop(0, n_chunks // 4, pairb, 0)
      pltpu.sync_copy(
          fl, out_hbm.at[pl.ds(wid * il_rows_w + hh * (il_rows_w // 2),
                               il_rows_w // 2)])
      return carry0

    lax.fori_loop(0, 2, half, 0)

  return reformat


@functools.lru_cache(maxsize=None)
def _build_main(B: int):
  rows_per_w = B // NW
  n_chunks = rows_per_w // R
  il_rows_w = n_chunks * CH_ROWS
  assert n_chunks % 2 == 0

  @functools.partial(
      pl.kernel,
      out_type=jax.ShapeDtypeStruct((B, 4 * D), jnp.float32),
      mesh=_mesh(),
      compiler_params=pltpu.CompilerParams(use_tc_tiling_on_sc=False,
                                           needs_layout_passes=False),
      scratch_types=[
          pltpu.VMEM((2, NPR, D), jnp.float32),
          pltpu.VMEM((2, CH_ROWS, 128), jnp.int32),
          pltpu.VMEM((2, 3, R), jnp.float32),
          pltpu.VMEM((2, 3, R), jnp.float32),
          pltpu.VMEM((3, D), jnp.float32),
          pltpu.VMEM((R, 4 * D), jnp.float32),
          pltpu.SemaphoreType.DMA,
          pltpu.SemaphoreType.DMA,
          pltpu.SemaphoreType.DMA,
          pltpu.SemaphoreType.DMA,
          pltpu.SemaphoreType.DMA,
          pltpu.SemaphoreType.DMA,
      ],
  )
  def main(il_hbm, tr_hbm, tg_hbm, td_hbm, ta_hbm, out_hbm,
           gbuf, ilist, ab, ab2, t0, outb,
           g0, g1, i0, i1, o0, o1):
    wid = lax.axis_index("c") * NS + lax.axis_index("s")
    gsem = (g0, g1)
    isem = (i0, i1)
    tbls = (tr_hbm, tg_hbm, td_hbm, ta_hbm)

    pltpu.sync_copy(tg_hbm.at[pl.ds(0, 1)], t0.at[pl.ds(0, 1)])
    pltpu.sync_copy(td_hbm.at[pl.ds(0, 1)], t0.at[pl.ds(1, 1)])
    pltpu.sync_copy(ta_hbm.at[pl.ds(0, 1)], t0.at[pl.ds(2, 1)])
    t0v = [[t0[fi, pl.ds(h * L, L)] for h in range(2)] for fi in range(3)]

    def stage_idx(c, s, sem):
      pltpu.async_copy(
          il_hbm.at[pl.ds(wid * il_rows_w + c * CH_ROWS, CH_ROWS)],
          ilist.at[s], sem)

    def wait_idx(s, sem):
      pltpu.make_async_copy(il_hbm.at[pl.ds(0, CH_ROWS)], ilist.at[s],
                            sem).wait()

    def fire_gathers(s, sem):
      for t, poff, goff, n in GPIECES:
        pltpu.async_copy(
            tbls[t].at[ilist.at[s, poff // 128, pl.ds(poff % 128, n)]],
            gbuf.at[s, pl.ds(goff, n)], sem)

    def wait_gathers(s, sem):
      pltpu.make_async_copy(ta_hbm.at[pl.ds(0, NPR)], gbuf.at[s], sem).wait()

    def phase(c, s):
      o = 1 - s
      base = wid * rows_per_w + c * R

      for fi, pseg, _, k in POOLED:
        for g in range(R // L):
          sl = pl.ds(g * L, L)
          cnt = jnp.zeros((L,), jnp.float32)
          for j in range(k):
            pos = pseg + j * R + g * L
            v = ilist[s, pos // 128, pl.ds(pos % 128, L)]
            cnt = cnt + jnp.where(v > 0, jnp.float32(1.0), jnp.float32(0.0))
          a = jnp.float32(1.0) / (cnt + jnp.float32(1e-8))
          ab[s, fi, sl] = a
          ab2[s, fi, sl] = (jnp.float32(k) - cnt) * a

      @pl.when(c + 1 < n_chunks)
      def _():
        wait_idx(o, isem[o])
        fire_gathers(o, gsem[o])

      wait_gathers(s, gsem[s])

      @pl.when(c + 2 < n_chunks)
      def _():
        stage_idx(c + 2, s, isem[s])

      @pl.when(c >= 1)
      def _():
        pltpu.make_async_copy(out_hbm.at[pl.ds(0, R)], outb, o0).wait()

      def row(r, carry):
        rsp = jnp.full((L,), r, jnp.int32)
        for h in range(2):
          sl = pl.ds(h * L, L)
          outb[r, pl.ds(h * L, L)] = gbuf[s, GSEG_RATE + r, sl]
        for fi, _, gseg, k in POOLED:
          av = plsc.load_gather(ab.at[s, fi], [rsp])
          bv = plsc.load_gather(ab2.at[s, fi], [rsp])
          for h in range(2):
            sl = pl.ds(h * L, L)
            acc = gbuf[s, gseg + r, sl]
            for j in range(1, k):
              acc = acc + gbuf[s, gseg + j * R + r, sl]
            outb[r, pl.ds((fi + 1) * D + h * L, L)] = (
                acc * av - t0v[fi][h] * bv)
        return carry

      lax.fori_loop(0, R, row, 0)

      pltpu.async_copy(outb, out_hbm.at[pl.ds(base, R)], o0)

    stage_idx(0, 0, isem[0])
    wait_idx(0, isem[0])
    fire_gathers(0, gsem[0])
    stage_idx(1, 1, isem[1])

    def pair(i, carry):
      phase(2 * i, 0)
      phase(2 * i + 1, 1)
      return carry

    lax.fori_loop(0, n_chunks // 2, pair, 0)

    pltpu.make_async_copy(out_hbm.at[pl.ds(0, R)], outb, o0).wait()

  return main


def kernel(rate_idx, genre_idx, director_idx, actors_idx,
           table_rate, table_genre, table_director, table_actor):
  B = rate_idx.shape[0]
  ilist = _build_reformat(B)(rate_idx, genre_idx, director_idx, actors_idx)
  return _build_main(B)(ilist, table_rate, table_genre, table_director,
                        table_actor)
